# baseline (speedup 1.0000x reference)
import functools

import jax
import jax.numpy as jnp
from jax import lax
from jax.experimental import pallas as pl
from jax.experimental.pallas import tpu as pltpu
from jax.experimental.pallas import tpu_sc as plsc

NODES = 1000000
D = 64
BB = 16384
FF = 26
TOTAL = BB * FF

_info = plsc.get_sparse_core_info()
_NC = _info.num_cores
_NW = _NC * _info.num_subcores

_NBLK = NODES // 128
_TRIPS = _NBLK // _NW
_LEFT = _NBLK - _TRIPS * _NW

_mesh = plsc.VectorSubcoreMesh(core_axis_name="c", subcore_axis_name="s")


def _transpose_block(inref, outref, n_srows, riota):
    def quad(q, carry):
        for dq in range(4):
            s = q * 4 + dq
            ce = jnp.full((16,), 2 * s, jnp.int32)
            co = jnp.full((16,), 2 * s + 1, jnp.int32)
            for kg in range(8):
                col = ce if kg < 4 else co
                v = plsc.load_gather(inref, [riota[kg % 4], col])
                outref[s, pl.ds(kg * 16, 16)] = v
        return carry

    lax.fori_loop(0, n_srows // 4, quad, 0)


@functools.partial(
    pl.kernel,
    mesh=_mesh,
    out_type=jax.ShapeDtypeStruct((NODES // 2, 128), jnp.float32),
    scratch_types=(
        [pltpu.VMEM((64, 128), jnp.float32) for _ in range(4)]
        + [pltpu.SemaphoreType.DMA for _ in range(4)]
    ),
    compiler_params=pltpu.CompilerParams(needs_layout_passes=False),
)
def _fmt_kernel(wt_hbm, tail_hbm, stag_hbm, inb0, inb1, outb0, outb1, is0, is1, os0, os1):
    wid = lax.axis_index("s") * _NC + lax.axis_index("c")
    inb = (inb0, inb1)
    outb = (outb0, outb1)
    isem = (is0, is1)
    osem = (os0, os1)
    riota = [lax.iota(jnp.int32, 16) + 16 * kg for kg in range(4)]

    def rt_of(i):
        return wid + _NW * i

    def fire_in(i, par):
        pltpu.async_copy(
            wt_hbm.at[:, pl.ds(rt_of(i) * 128, 128)], inb[par], isem[par]
        )

    def drain(sem, par, buf):
        pltpu.make_async_copy(
            wt_hbm.at[:, pl.ds(0, 128)], buf[par], sem[par]
        ).wait()

    fire_in(0, 0)
    fire_in(1, 1)

    def pair(g, carry):
        for par in range(2):
            i = 2 * g + par
            drain(isem, par, inb)

            @pl.when(g >= 1)
            def _():
                drain(osem, par, outb)

            _transpose_block(inb[par], outb[par], 64, riota)
            pltpu.async_copy(
                outb[par], stag_hbm.at[pl.ds(rt_of(i) * 64, 64)], osem[par]
            )
            fire_in(i + 2, par)
        return carry

    lax.fori_loop(0, _TRIPS // 2 - 1, pair, 0)

    for par in range(2):
        i = _TRIPS - 2 + par
        drain(isem, par, inb)
        drain(osem, par, outb)
        _transpose_block(inb[par], outb[par], 64, riota)
        pltpu.async_copy(
            outb[par], stag_hbm.at[pl.ds(rt_of(i) * 64, 64)], osem[par]
        )
    for par in range(2):
        drain(osem, par, outb)

    for k in range(_LEFT):

        @pl.when(wid == k)
        def _():
            rt = _TRIPS * _NW + k
            pltpu.sync_copy(wt_hbm.at[:, pl.ds(rt * 128, 128)], inb0)
            _transpose_block(inb0, outb0, 64, riota)
            pltpu.sync_copy(outb0, stag_hbm.at[pl.ds(rt * 64, 64)])

    @pl.when(wid == _LEFT)
    def _():
        base = (_TRIPS * _NW + _LEFT) * 128
        pltpu.sync_copy(tail_hbm, inb0.at[pl.ds(0, 32)])
        pltpu.sync_copy(inb0.at[pl.ds(0, 32)], stag_hbm.at[pl.ds(base // 2, 32)])


_PW = TOTAL // _NW
_BC = 16
_CR = _BC * FF
_NCH = _PW // _CR


@functools.partial(
    pl.kernel,
    mesh=_mesh,
    out_type=jax.ShapeDtypeStruct((FF, 8, BB // 128, 8, 128), jnp.float32),
    scratch_types=(
        [pltpu.VMEM((_PW,), jnp.int32)]
        + [pltpu.VMEM((_CR, D), jnp.float32) for _ in range(2)]
        + [pltpu.VMEM((FF, 8, 8, _BC), jnp.float32) for _ in range(2)]
        + [pltpu.SemaphoreType.DMA for _ in range(4)]
    ),
    compiler_params=pltpu.CompilerParams(
        use_tc_tiling_on_sc=False, needs_layout_passes=False
    ),
)
def _gather_kernel(idx_hbm, stag_hbm, out_hbm, idx_v, r0, r1, v0, v1, g0, g1, w0, w1):
    wid = lax.axis_index("s") * _NC + lax.axis_index("c")
    base = wid * _PW
    rows = (r0, r1)
    vbuf = (v0, v1)
    gsem = (g0, g1)
    wsem = (w0, w1)
    i26 = lax.iota(jnp.int32, 16) * FF

    pltpu.sync_copy(idx_hbm.at[pl.ds(base, _PW)], idx_v)

    def fire_gathers(c, par):
        off = c * _CR
        for j, (o, n) in enumerate(((0, 128), (128, 128), (256, 128), (384, 32))):
            pltpu.async_copy(
                stag_hbm.at[idx_v.at[pl.ds(off + o, n)]],
                rows[par].at[pl.ds(o, n)],
                gsem[par],
            )

    def drain_gathers(par):
        for o, n in ((0, 128), (128, 128), (256, 128), (384, 32)):
            pltpu.make_async_copy(
                stag_hbm.at[idx_v.at[pl.ds(0, n)]],
                rows[par].at[pl.ds(o, n)],
                gsem[par],
            ).wait()

    def drain_write(par):
        pltpu.make_async_copy(
            out_hbm.at[:, :, 0, :, pl.ds(0, _BC)], vbuf[par], wsem[par]
        ).wait()

    def transpose_chunk(par):
        def fbody(f, carry):
            rowi = i26 + f
            for jt in range(8):
                for j in range(8):
                    colv = jnp.full((16,), jt * 8 + j, jnp.int32)
                    v = plsc.load_gather(rows[par], [rowi, colv])
                    vbuf[par][f, jt, j, :] = v
            return carry

        lax.fori_loop(0, FF, fbody, 0)

    fire_gathers(0, 0)
    fire_gathers(1, 1)

    def visit(g, carry):
        for par in range(2):
            c = 2 * g + par
            drain_gathers(par)

            @pl.when(c >= 2)
            def _():
                drain_write(par)

            transpose_chunk(par)
            b0 = wid * 512 + c * _BC
            bt = b0 // 128
            bo = b0 % 128
            pltpu.async_copy(
                vbuf[par],
                out_hbm.at[:, :, bt, :, pl.ds(bo, _BC)],
                wsem[par],
            )
            cn = jnp.minimum(c + 2, _NCH - 1)
            fire_gathers(cn, par)
        return carry

    lax.fori_loop(0, _NCH // 2 - 1, visit, 0)

    for par in range(2):
        c = _NCH - 2 + par
        drain_gathers(par)
        drain_write(par)
        transpose_chunk(par)
        b0 = wid * 512 + c * _BC
        pltpu.async_copy(
            vbuf[par],
            out_hbm.at[:, :, b0 // 128, :, pl.ds(b0 % 128, _BC)],
            wsem[par],
        )
    for par in range(2):
        drain_write(par)


def kernel(x, m, W):
    idx = x.reshape(TOTAL).astype(jnp.int32)
    tail = W[_NBLK * 128:].reshape(32, 128)
    stag = _fmt_kernel(W.T, tail).reshape(NODES, D)
    o5 = _gather_kernel(idx, stag)
    return o5.transpose(2, 4, 0, 1, 3).reshape(BB, FF, D)


# speedup vs baseline: 1.8084x; 1.8084x over previous
import functools

import jax
import jax.numpy as jnp
from jax import lax
from jax.experimental import pallas as pl
from jax.experimental.pallas import tpu as pltpu
from jax.experimental.pallas import tpu_sc as plsc

NODES = 1000000
D = 64
BB = 16384
FF = 26
TOTAL = BB * FF

_info = plsc.get_sparse_core_info()
_NC = _info.num_cores
_NW = _NC * _info.num_subcores

_NBLK = NODES // 128
_TRIPS = _NBLK // _NW
_LEFT = _NBLK - _TRIPS * _NW

_mesh = plsc.VectorSubcoreMesh(core_axis_name="c", subcore_axis_name="s")


def _transpose_block(inref, outref, n_srows, riota):
    @plsc.parallel_loop(0, n_srows, unroll=4)
    def _srow(s):
        ce = jnp.full((16,), 2 * s, jnp.int32)
        co = jnp.full((16,), 2 * s + 1, jnp.int32)
        for kg in range(8):
            col = ce if kg < 4 else co
            v = plsc.load_gather(inref, [riota[kg % 4], col])
            outref[s, pl.ds(kg * 16, 16)] = v


@functools.partial(
    pl.kernel,
    mesh=_mesh,
    out_type=jax.ShapeDtypeStruct((NODES // 2, 128), jnp.float32),
    scratch_types=(
        [pltpu.VMEM((64, 128), jnp.float32) for _ in range(4)]
        + [pltpu.SemaphoreType.DMA for _ in range(4)]
    ),
    compiler_params=pltpu.CompilerParams(needs_layout_passes=False),
)
def _fmt_kernel(wt_hbm, tail_hbm, stag_hbm, inb0, inb1, outb0, outb1, is0, is1, os0, os1):
    wid = lax.axis_index("s") * _NC + lax.axis_index("c")
    inb = (inb0, inb1)
    outb = (outb0, outb1)
    isem = (is0, is1)
    osem = (os0, os1)
    riota = [lax.iota(jnp.int32, 16) + 16 * kg for kg in range(4)]

    def rt_of(i):
        return wid + _NW * i

    def fire_in(i, par):
        pltpu.async_copy(
            wt_hbm.at[:, pl.ds(rt_of(i) * 128, 128)], inb[par], isem[par]
        )

    def drain(sem, par, buf):
        pltpu.make_async_copy(
            wt_hbm.at[:, pl.ds(0, 128)], buf[par], sem[par]
        ).wait()

    fire_in(0, 0)
    fire_in(1, 1)

    def pair(g, carry):
        for par in range(2):
            i = 2 * g + par
            drain(isem, par, inb)

            @pl.when(g >= 1)
            def _():
                drain(osem, par, outb)

            _transpose_block(inb[par], outb[par], 64, riota)
            pltpu.async_copy(
                outb[par], stag_hbm.at[pl.ds(rt_of(i) * 64, 64)], osem[par]
            )
            fire_in(i + 2, par)
        return carry

    lax.fori_loop(0, _TRIPS // 2 - 1, pair, 0)

    for par in range(2):
        i = _TRIPS - 2 + par
        drain(isem, par, inb)
        drain(osem, par, outb)
        _transpose_block(inb[par], outb[par], 64, riota)
        pltpu.async_copy(
            outb[par], stag_hbm.at[pl.ds(rt_of(i) * 64, 64)], osem[par]
        )
    for par in range(2):
        drain(osem, par, outb)

    for k in range(_LEFT):

        @pl.when(wid == k)
        def _():
            rt = _TRIPS * _NW + k
            pltpu.sync_copy(wt_hbm.at[:, pl.ds(rt * 128, 128)], inb0)
            _transpose_block(inb0, outb0, 64, riota)
            pltpu.sync_copy(outb0, stag_hbm.at[pl.ds(rt * 64, 64)])

    @pl.when(wid == _LEFT)
    def _():
        base = (_TRIPS * _NW + _LEFT) * 128
        pltpu.sync_copy(tail_hbm, inb0.at[pl.ds(0, 32)])
        pltpu.sync_copy(inb0.at[pl.ds(0, 32)], stag_hbm.at[pl.ds(base // 2, 32)])


_PW = TOTAL // _NW
_BC = 16
_CR = _BC * FF
_NCH = _PW // _CR


@functools.partial(
    pl.kernel,
    mesh=_mesh,
    out_type=jax.ShapeDtypeStruct((FF, 8, BB // 128, 8, 128), jnp.float32),
    scratch_types=(
        [pltpu.VMEM((_PW,), jnp.int32)]
        + [pltpu.VMEM((_CR, D), jnp.float32) for _ in range(2)]
        + [pltpu.VMEM((FF, 8, 8, _BC), jnp.float32) for _ in range(2)]
        + [pltpu.SemaphoreType.DMA for _ in range(4)]
    ),
    compiler_params=pltpu.CompilerParams(
        use_tc_tiling_on_sc=False, needs_layout_passes=False
    ),
)
def _gather_kernel(idx_hbm, stag_hbm, out_hbm, idx_v, r0, r1, v0, v1, g0, g1, w0, w1):
    wid = lax.axis_index("s") * _NC + lax.axis_index("c")
    base = wid * _PW
    rows = (r0, r1)
    vbuf = (v0, v1)
    gsem = (g0, g1)
    wsem = (w0, w1)
    i26 = lax.iota(jnp.int32, 16) * FF

    pltpu.sync_copy(idx_hbm.at[pl.ds(base, _PW)], idx_v)

    def fire_gathers(c, par):
        off = c * _CR
        for j, (o, n) in enumerate(((0, 128), (128, 128), (256, 128), (384, 32))):
            pltpu.async_copy(
                stag_hbm.at[idx_v.at[pl.ds(off + o, n)]],
                rows[par].at[pl.ds(o, n)],
                gsem[par],
            )

    def drain_gathers(par):
        for o, n in ((0, 128), (128, 128), (256, 128), (384, 32)):
            pltpu.make_async_copy(
                stag_hbm.at[idx_v.at[pl.ds(0, n)]],
                rows[par].at[pl.ds(o, n)],
                gsem[par],
            ).wait()

    def drain_write(par):
        pltpu.make_async_copy(
            out_hbm.at[:, :, 0, :, pl.ds(0, _BC)], vbuf[par], wsem[par]
        ).wait()

    def transpose_chunk(par):
        @plsc.parallel_loop(0, FF, unroll=2)
        def _fbody(f):
            rowi = i26 + f
            for jt in range(8):
                for j in range(8):
                    colv = jnp.full((16,), jt * 8 + j, jnp.int32)
                    v = plsc.load_gather(rows[par], [rowi, colv])
                    vbuf[par][f, jt, j, :] = v

    fire_gathers(0, 0)
    fire_gathers(1, 1)

    def visit(g, carry):
        for par in range(2):
            c = 2 * g + par
            drain_gathers(par)

            @pl.when(c >= 2)
            def _():
                drain_write(par)

            transpose_chunk(par)
            b0 = wid * 512 + c * _BC
            bt = b0 // 128
            bo = b0 % 128
            pltpu.async_copy(
                vbuf[par],
                out_hbm.at[:, :, bt, :, pl.ds(bo, _BC)],
                wsem[par],
            )
            cn = jnp.minimum(c + 2, _NCH - 1)
            fire_gathers(cn, par)
        return carry

    lax.fori_loop(0, _NCH // 2 - 1, visit, 0)

    for par in range(2):
        c = _NCH - 2 + par
        drain_gathers(par)
        drain_write(par)
        transpose_chunk(par)
        b0 = wid * 512 + c * _BC
        pltpu.async_copy(
            vbuf[par],
            out_hbm.at[:, :, b0 // 128, :, pl.ds(b0 % 128, _BC)],
            wsem[par],
        )
    for par in range(2):
        drain_write(par)


def kernel(x, m, W):
    idx = x.reshape(TOTAL).astype(jnp.int32)
    tail = W[_NBLK * 128:].reshape(32, 128)
    stag = _fmt_kernel(W.T, tail).reshape(NODES, D)
    o5 = _gather_kernel(idx, stag)
    return o5.transpose(2, 4, 0, 1, 3).reshape(BB, FF, D)
